# baseline (speedup 1.0000x reference)
import jax
import jax.numpy as jnp
from jax import lax
from jax.experimental import pallas as pl
from jax.experimental.pallas import tpu as pltpu
from jax.experimental.pallas import tpu_sc as plsc

P = 100000
C = 20000
D = 128
B = 160
NPB = P // B
NCB = C // B
NW = 32
NG = D // 32
DW = D // 2
M_HI = -65536


def _sc_kernel(fp_t, ipc_t, pc_t, ci_t, cp_t,
               fp_b, ipc_b, pcb_b, cp_b, ind_b,
               patent_f32, company_f32, out,
               i00, i01, i10, i11, r00, r01, r10, r11,
               acc_v, base_v, out_v,
               is00, is01, is10, is11, gs00, gs01, gs10, gs11,
               bsem, outsem):
    nc = 2
    wid = lax.axis_index("s") * nc + lax.axis_index("c")
    ibuf = ((i00, i01), (i10, i11))
    isem = ((is00, is01), (is10, is11))
    rbuf = ((r00, r01), (r10, r11))
    gsem = ((gs00, gs01), (gs10, gs11))

    def pair_pass(src0, src1, first):
        def row(i, _):
            for v in range(NG):
                slw = pl.ds(v * 16, 16)
                w0 = src0[i, slw]
                w1 = src1[i, slw]
                bc = lambda x: lax.bitcast_convert_type(x, jnp.float32)
                lo = bc(w0 << 16) + bc(w1 << 16)
                hi = bc(w0 & M_HI) + bc(w1 & M_HI)
                sa = pl.ds(v * 32, 16)
                sb = pl.ds(v * 32 + 16, 16)
                if first:
                    acc_v[i, sa] = lo
                    acc_v[i, sb] = hi
                else:
                    plsc.addupdate(acc_v.at[i, sa], lo)
                    plsc.addupdate(acc_v.at[i, sb], hi)
            return 0
        lax.fori_loop(0, B, row, 0)

    def final_rows(scale):
        def row(i, _):
            for v in range(D // 16):
                sl = pl.ds(v * 16, 16)
                out_v[i, sl] = base_v[i, sl] + acc_v[i, sl] * scale
            return 0
        lax.fori_loop(0, B, row, 0)

    def phase(nb, nt, slots, base_tab, out_off, scale, first_phase):
        n = len(slots)
        np_ = n // 2

        def issue_idx(p, base, s):
            for k in range(2):
                arr, _, off = slots[2 * p + k]
                pltpu.async_copy(arr.at[pl.ds(off + base, B)],
                                 ibuf[s][k], isem[s][k])

        def wait_idx(p, base, s):
            for k in range(2):
                arr, _, off = slots[2 * p + k]
                pltpu.make_async_copy(arr.at[pl.ds(off + base, B)],
                                      ibuf[s][k], isem[s][k]).wait()

        def issue_gather(p, s):
            for k in range(2):
                _, tab, _ = slots[2 * p + k]
                pltpu.async_copy(tab.at[ibuf[s][k]], rbuf[s][k], gsem[s][k])

        def wait_gather(p, s):
            for k in range(2):
                _, tab, _ = slots[2 * p + k]
                pltpu.make_async_copy(tab.at[ibuf[s][k]], rbuf[s][k],
                                      gsem[s][k]).wait()

        @pl.when(wid < nb)
        def _():
            issue_idx(0, wid * B, 0)

        def block(t, _):
            b = wid + t * NW

            @pl.when(b < nb)
            def _():
                base = b * B
                pltpu.async_copy(base_tab.at[pl.ds(base, B)], base_v, bsem)
                for p in range(np_):
                    s = p % 2
                    wait_idx(p, base, s)
                    issue_gather(p, s)
                    if p == 0:
                        issue_idx(1, base, 1)
                    else:
                        wait_gather(p - 1, 1 - s)
                        if p + 1 < np_:
                            issue_idx(p + 1, base, (p + 1) % 2)
                        pair_pass(rbuf[1 - s][0], rbuf[1 - s][1], p - 1 == 0)
                sl = (np_ - 1) % 2
                wait_gather(np_ - 1, sl)

                @pl.when(b + NW < nb)
                def _():
                    issue_idx(0, base + NW * B, 0)

                pair_pass(rbuf[sl][0], rbuf[sl][1], False)
                pltpu.make_async_copy(base_tab.at[pl.ds(base, B)], base_v, bsem).wait()
                if first_phase:
                    @pl.when(t > 0)
                    def _():
                        pltpu.make_async_copy(out_v, out.at[pl.ds(0, B)], outsem).wait()
                else:
                    pltpu.make_async_copy(out_v, out.at[pl.ds(0, B)], outsem).wait()
                final_rows(scale)
                pltpu.async_copy(out_v, out.at[pl.ds(out_off + base, B)], outsem)
            return 0

        lax.fori_loop(0, nt, block, 0)

    p_slots = ([(fp_t, fp_b, j * P) for j in range(4)]
               + [(ipc_t, ipc_b, j * P) for j in range(2)]
               + [(pc_t, pcb_b, j * P) for j in range(2)])
    c_slots = ([(ci_t, ind_b, j * C) for j in range(2)]
               + [(cp_t, cp_b, j * C) for j in range(16)])

    phase(NPB, pl.cdiv(NPB, NW), p_slots, patent_f32, 0, 0.125, True)
    phase(NCB, pl.cdiv(NCB, NW), c_slots, company_f32, P, 1.0 / 18.0, False)
    pltpu.make_async_copy(out_v, out.at[pl.ds(0, B)], outsem).wait()


def _pack_bf16(t):
    v = t.shape[0]
    b16 = (t.reshape(v, D // 32, 2, 16).transpose(0, 1, 3, 2)
           .reshape(v, DW, 2).astype(jnp.bfloat16))
    return jax.lax.bitcast_convert_type(b16, jnp.int32)


def kernel(pid_fp_idx, pid_ipc_idx, patent_company_idx, company_industry_idx,
           company_patent_idx, company_table, patent_table, fp_table,
           ipc_table, industry_table):
    fp_t = pid_fp_idx.T.reshape(-1).astype(jnp.int32)
    ipc_t = pid_ipc_idx.T.reshape(-1).astype(jnp.int32)
    pc_t = patent_company_idx.T.reshape(-1).astype(jnp.int32)
    ci_t = company_industry_idx.T.reshape(-1).astype(jnp.int32)
    cp_t = company_patent_idx.T.reshape(-1).astype(jnp.int32)

    fp_b = _pack_bf16(fp_table)
    ipc_b = _pack_bf16(ipc_table)
    pcb_b = _pack_bf16(company_table)
    cp_b = _pack_bf16(patent_table)
    ind_b = _pack_bf16(industry_table)

    mesh = plsc.VectorSubcoreMesh(core_axis_name="c", subcore_axis_name="s")
    run = pl.kernel(
        _sc_kernel,
        out_type=jax.ShapeDtypeStruct((P + C, D), jnp.float32),
        mesh=mesh,
        compiler_params=pltpu.CompilerParams(use_tc_tiling_on_sc=False),
        scratch_types=[
            pltpu.VMEM((B,), jnp.int32),
            pltpu.VMEM((B,), jnp.int32),
            pltpu.VMEM((B,), jnp.int32),
            pltpu.VMEM((B,), jnp.int32),
            pltpu.VMEM((B, DW), jnp.int32),
            pltpu.VMEM((B, DW), jnp.int32),
            pltpu.VMEM((B, DW), jnp.int32),
            pltpu.VMEM((B, DW), jnp.int32),
            pltpu.VMEM((B, D), jnp.float32),
            pltpu.VMEM((B, D), jnp.float32),
            pltpu.VMEM((B, D), jnp.float32),
            pltpu.SemaphoreType.DMA,
            pltpu.SemaphoreType.DMA,
            pltpu.SemaphoreType.DMA,
            pltpu.SemaphoreType.DMA,
            pltpu.SemaphoreType.DMA,
            pltpu.SemaphoreType.DMA,
            pltpu.SemaphoreType.DMA,
            pltpu.SemaphoreType.DMA,
            pltpu.SemaphoreType.DMA,
            pltpu.SemaphoreType.DMA,
        ],
    )
    return run(fp_t, ipc_t, pc_t, ci_t, cp_t,
               fp_b, ipc_b, pcb_b, cp_b, ind_b,
               patent_table, company_table)


# speedup vs baseline: 1.8868x; 1.8868x over previous
import jax
import jax.numpy as jnp
from jax import lax
from jax.experimental import pallas as pl
from jax.experimental.pallas import tpu as pltpu
from jax.experimental.pallas import tpu_sc as plsc

P = 100000
C = 20000
D = 128
B = 160
NPB = P // B
NCB = C // B
NW = 32
NV = D // 16


def _sc_kernel(fp_t, ipc_t, pc_t, ci_t, cp_t,
               company_table, patent_table, fp_table, ipc_table,
               industry_table, out,
               i_first, i0, i1, r0, r1, accA, accB, baseA, baseB,
               isem_f, isem0, isem1, gsem0, gsem1, sema,
               bsemA, bsemB, outsemA, outsemB):
    nc = 2
    wid = lax.axis_index("s") * nc + lax.axis_index("c")
    i_cyc = (i0, i1)
    isems = (isem0, isem1)
    rbuf = (r0, r1)
    gsems = (gsem0, gsem1)

    def accumulate_rows(acc_v, src):
        def row(i4, _):
            for u in range(4):
                i = i4 * 4 + u
                for v in range(NV):
                    sl = pl.ds(v * 16, 16)
                    plsc.addupdate(acc_v.at[i, sl], src[i, sl])
            return 0
        lax.fori_loop(0, B // 4, row, 0)

    def final_rows(acc_v, base_v, scale):
        def row(i2, _):
            for u in range(2):
                i = i2 * 2 + u
                for v in range(NV):
                    sl = pl.ds(v * 16, 16)
                    acc_v[i, sl] = base_v[i, sl] + acc_v[i, sl] * scale
            return 0
        lax.fori_loop(0, B // 2, row, 0)

    def phase(nb, nt, slots, base_tab, out_off, scale, first_phase):
        n = len(slots)
        arr0, tab0, off0 = slots[0]
        accs = (accA, accB)
        bases = (baseA, baseB)
        bsems = (bsemA, bsemB)
        osems = (outsemA, outsemB)

        @pl.when(wid < nb)
        def _():
            pltpu.async_copy(arr0.at[pl.ds(off0 + wid * B, B)], i_first, isem_f)

        def finalize(tp, acc_v, base_v, bsem, osem):
            base = (wid + tp * NW) * B
            pltpu.make_async_copy(base_tab.at[pl.ds(base, B)], base_v, bsem).wait()
            final_rows(acc_v, base_v, scale)
            pltpu.async_copy(acc_v, out.at[pl.ds(out_off + base, B)], osem)

        def block(t, par):
            b = wid + t * NW
            acc_v = accs[par]
            base_v = bases[par]

            def finalize_prev():
                finalize(t - 1, accs[1 - par], bases[1 - par],
                         bsems[1 - par], osems[1 - par])

            @pl.when(b < nb)
            def _():
                base = b * B
                if first_phase:
                    @pl.when(t > 1)
                    def _():
                        pltpu.make_async_copy(acc_v, out.at[pl.ds(0, B)], osems[par]).wait()
                else:
                    pltpu.make_async_copy(acc_v, out.at[pl.ds(0, B)], osems[par]).wait()
                pltpu.async_copy(base_tab.at[pl.ds(base, B)], base_v, bsems[par])
                pltpu.make_async_copy(arr0.at[pl.ds(off0 + base, B)], i_first, isem_f).wait()
                pltpu.async_copy(tab0.at[i_first], acc_v, sema)
                arr1, _, offs1 = slots[1]
                pltpu.async_copy(arr1.at[pl.ds(offs1 + base, B)], i_cyc[1], isems[1])

                for k in range(1, n):
                    kb = k % 2
                    arrk, tabk, offk = slots[k]
                    pltpu.make_async_copy(
                        arrk.at[pl.ds(offk + base, B)], i_cyc[kb], isems[kb]).wait()
                    pltpu.async_copy(tabk.at[i_cyc[kb]], rbuf[kb], gsems[kb])
                    if k == 1:
                        pltpu.async_copy(
                            slots[2][0].at[pl.ds(slots[2][2] + base, B)],
                            i_cyc[0], isems[0])
                        pltpu.make_async_copy(tab0.at[i_first], acc_v, sema).wait()
                        @pl.when((t > 0) & (b - NW < nb))
                        def _():
                            finalize_prev()
                    else:
                        pkb = (k - 1) % 2
                        pltpu.make_async_copy(
                            slots[k - 1][1].at[i_cyc[pkb]], rbuf[pkb],
                            gsems[pkb]).wait()
                        if k + 1 < n:
                            arrn, _, offn = slots[k + 1]
                            pltpu.async_copy(
                                arrn.at[pl.ds(offn + base, B)],
                                i_cyc[(k + 1) % 2], isems[(k + 1) % 2])
                        else:
                            @pl.when(b + NW < nb)
                            def _():
                                pltpu.async_copy(
                                    arr0.at[pl.ds(off0 + (base + NW * B), B)],
                                    i_first, isem_f)
                        accumulate_rows(acc_v, rbuf[pkb])

                lkb = (n - 1) % 2
                pltpu.make_async_copy(
                    slots[n - 1][1].at[i_cyc[lkb]], rbuf[lkb], gsems[lkb]).wait()
                accumulate_rows(acc_v, rbuf[lkb])

            @pl.when((b >= nb) & (t > 0) & (b - NW < nb))
            def _():
                finalize_prev()

        def halfstep(j, _):
            block(j * 2, 0)
            block(j * 2 + 1, 1)
            return 0

        lax.fori_loop(0, nt // 2, halfstep, 0)

        lastp = (nt - 1) % 2

        @pl.when(wid + (nt - 1) * NW < nb)
        def _():
            finalize(nt - 1, accs[lastp], bases[lastp], bsems[lastp], osems[lastp])

    p_slots = ([(fp_t, fp_table, j * P) for j in range(4)]
               + [(ipc_t, ipc_table, j * P) for j in range(2)]
               + [(pc_t, company_table, j * P) for j in range(2)])
    c_slots = ([(ci_t, industry_table, j * C) for j in range(2)]
               + [(cp_t, patent_table, j * C) for j in range(16)])

    phase(NPB, pl.cdiv(NPB, NW), p_slots, patent_table, 0, 0.125, True)
    phase(NCB, pl.cdiv(NCB, NW), c_slots, company_table, P, 1.0 / 18.0, False)
    pltpu.make_async_copy(accA, out.at[pl.ds(0, B)], outsemA).wait()
    pltpu.make_async_copy(accB, out.at[pl.ds(0, B)], outsemB).wait()


def kernel(pid_fp_idx, pid_ipc_idx, patent_company_idx, company_industry_idx,
           company_patent_idx, company_table, patent_table, fp_table,
           ipc_table, industry_table):
    fp_t = pid_fp_idx.T.reshape(-1).astype(jnp.int32)
    ipc_t = pid_ipc_idx.T.reshape(-1).astype(jnp.int32)
    pc_t = patent_company_idx.T.reshape(-1).astype(jnp.int32)
    ci_t = company_industry_idx.T.reshape(-1).astype(jnp.int32)
    cp_t = company_patent_idx.T.reshape(-1).astype(jnp.int32)

    mesh = plsc.VectorSubcoreMesh(core_axis_name="c", subcore_axis_name="s")
    run = pl.kernel(
        _sc_kernel,
        out_type=jax.ShapeDtypeStruct((P + C, D), jnp.float32),
        mesh=mesh,
        scratch_types=[
            pltpu.VMEM((B,), jnp.int32),
            pltpu.VMEM((B,), jnp.int32),
            pltpu.VMEM((B,), jnp.int32),
            pltpu.VMEM((B, D), jnp.float32),
            pltpu.VMEM((B, D), jnp.float32),
            pltpu.VMEM((B, D), jnp.float32),
            pltpu.VMEM((B, D), jnp.float32),
            pltpu.VMEM((B, D), jnp.float32),
            pltpu.VMEM((B, D), jnp.float32),
            pltpu.SemaphoreType.DMA,
            pltpu.SemaphoreType.DMA,
            pltpu.SemaphoreType.DMA,
            pltpu.SemaphoreType.DMA,
            pltpu.SemaphoreType.DMA,
            pltpu.SemaphoreType.DMA,
            pltpu.SemaphoreType.DMA,
            pltpu.SemaphoreType.DMA,
            pltpu.SemaphoreType.DMA,
            pltpu.SemaphoreType.DMA,
        ],
    )
    return run(fp_t, ipc_t, pc_t, ci_t, cp_t, company_table, patent_table,
               fp_table, ipc_table, industry_table)


# speedup vs baseline: 1.9422x; 1.0294x over previous
import jax
import jax.numpy as jnp
from jax import lax
from jax.experimental import pallas as pl
from jax.experimental.pallas import tpu as pltpu
from jax.experimental.pallas import tpu_sc as plsc

P = 100000
C = 20000
D = 128
B = 160
NPB = P // B
NCB = C // B
NW = 32
NV = D // 16


def _sc_kernel(fp_t, ipc_t, pc_t, ci_t, cp_t,
               company_table, patent_table, fp_table, ipc_table,
               industry_table, out,
               i_first, i0, i1, r0, r1, acc_v, base_v,
               isem_f, isem0, isem1, gsem0, gsem1, sema, bsem, outsem):
    nc = 2
    wid = lax.axis_index("s") * nc + lax.axis_index("c")
    i_cyc = (i0, i1)
    isems = (isem0, isem1)
    rbuf = (r0, r1)
    gsems = (gsem0, gsem1)

    def accumulate_rows(src):
        def row(i2, _):
            for u in range(2):
                i = i2 * 2 + u
                for v in range(NV):
                    sl = pl.ds(v * 16, 16)
                    plsc.addupdate(acc_v.at[i, sl], src[i, sl])
            return 0
        lax.fori_loop(0, B // 2, row, 0)

    def final_rows(src, scale):
        def row(i, _):
            for v in range(NV):
                sl = pl.ds(v * 16, 16)
                acc_v[i, sl] = base_v[i, sl] + (acc_v[i, sl] + src[i, sl]) * scale
            return 0
        lax.fori_loop(0, B, row, 0)

    def phase(nb, nt, slots, base_tab, out_off, scale, first_phase):
        n = len(slots)

        arr0, _, off0 = slots[0]

        @pl.when(wid < nb)
        def _():
            pltpu.async_copy(arr0.at[pl.ds(off0 + wid * B, B)], i_first, isem_f)

        def block(t, _):
            b = wid + t * NW

            @pl.when(b < nb)
            def _():
                base = b * B
                pltpu.async_copy(base_tab.at[pl.ds(base, B)], base_v, bsem)
                if first_phase:
                    @pl.when(t > 0)
                    def _():
                        pltpu.make_async_copy(acc_v, out.at[pl.ds(0, B)], outsem).wait()
                else:
                    pltpu.make_async_copy(acc_v, out.at[pl.ds(0, B)], outsem).wait()
                pltpu.make_async_copy(arr0.at[pl.ds(off0 + base, B)], i_first, isem_f).wait()
                _, tab0, _ = slots[0]
                pltpu.async_copy(tab0.at[i_first], acc_v, sema)
                arr1, _, offs1 = slots[1]
                pltpu.async_copy(arr1.at[pl.ds(offs1 + base, B)], i_cyc[1], isems[1])

                for k in range(1, n):
                    kb = k % 2
                    arrk, tabk, offk = slots[k]
                    pltpu.make_async_copy(
                        arrk.at[pl.ds(offk + base, B)], i_cyc[kb], isems[kb]).wait()
                    pltpu.async_copy(tabk.at[i_cyc[kb]], rbuf[kb], gsems[kb])
                    if k == 1:
                        pltpu.async_copy(
                            slots[2][0].at[pl.ds(slots[2][2] + base, B)],
                            i_cyc[0], isems[0])
                        pltpu.make_async_copy(tab0.at[i_first], acc_v, sema).wait()
                    else:
                        pkb = (k - 1) % 2
                        pltpu.make_async_copy(
                            slots[k - 1][1].at[i_cyc[pkb]], rbuf[pkb],
                            gsems[pkb]).wait()
                        if k + 1 < n:
                            arrn, _, offn = slots[k + 1]
                            pltpu.async_copy(
                                arrn.at[pl.ds(offn + base, B)],
                                i_cyc[(k + 1) % 2], isems[(k + 1) % 2])
                        else:
                            @pl.when(b + NW < nb)
                            def _():
                                pltpu.async_copy(
                                    arr0.at[pl.ds(off0 + (base + NW * B), B)],
                                    i_first, isem_f)
                        accumulate_rows(rbuf[pkb])

                lkb = (n - 1) % 2
                pltpu.make_async_copy(
                    slots[n - 1][1].at[i_cyc[lkb]], rbuf[lkb], gsems[lkb]).wait()
                pltpu.make_async_copy(base_tab.at[pl.ds(base, B)], base_v, bsem).wait()
                final_rows(rbuf[lkb], scale)
                pltpu.async_copy(acc_v, out.at[pl.ds(out_off + base, B)], outsem)
            return 0

        lax.fori_loop(0, nt, block, 0)

    p_slots = ([(fp_t, fp_table, j * P) for j in range(4)]
               + [(ipc_t, ipc_table, j * P) for j in range(2)]
               + [(pc_t, company_table, j * P) for j in range(2)])
    c_slots = ([(ci_t, industry_table, j * C) for j in range(2)]
               + [(cp_t, patent_table, j * C) for j in range(16)])

    phase(NPB, pl.cdiv(NPB, NW), p_slots, patent_table, 0, 0.125, True)
    phase(NCB, pl.cdiv(NCB, NW), c_slots, company_table, P, 1.0 / 18.0, False)
    pltpu.make_async_copy(acc_v, out.at[pl.ds(0, B)], outsem).wait()


def kernel(pid_fp_idx, pid_ipc_idx, patent_company_idx, company_industry_idx,
           company_patent_idx, company_table, patent_table, fp_table,
           ipc_table, industry_table):
    fp_t = pid_fp_idx.T.reshape(-1).astype(jnp.int32)
    ipc_t = pid_ipc_idx.T.reshape(-1).astype(jnp.int32)
    pc_t = patent_company_idx.T.reshape(-1).astype(jnp.int32)
    ci_t = company_industry_idx.T.reshape(-1).astype(jnp.int32)
    cp_t = company_patent_idx.T.reshape(-1).astype(jnp.int32)

    mesh = plsc.VectorSubcoreMesh(core_axis_name="c", subcore_axis_name="s")
    run = pl.kernel(
        _sc_kernel,
        out_type=jax.ShapeDtypeStruct((P + C, D), jnp.float32),
        mesh=mesh,
        scratch_types=[
            pltpu.VMEM((B,), jnp.int32),
            pltpu.VMEM((B,), jnp.int32),
            pltpu.VMEM((B,), jnp.int32),
            pltpu.VMEM((B, D), jnp.float32),
            pltpu.VMEM((B, D), jnp.float32),
            pltpu.VMEM((B, D), jnp.float32),
            pltpu.VMEM((B, D), jnp.float32),
            pltpu.SemaphoreType.DMA,
            pltpu.SemaphoreType.DMA,
            pltpu.SemaphoreType.DMA,
            pltpu.SemaphoreType.DMA,
            pltpu.SemaphoreType.DMA,
            pltpu.SemaphoreType.DMA,
            pltpu.SemaphoreType.DMA,
            pltpu.SemaphoreType.DMA,
        ],
    )
    return run(fp_t, ipc_t, pc_t, ci_t, cp_t, company_table, patent_table,
               fp_table, ipc_table, industry_table)
